# Initial kernel scaffold; baseline (speedup 1.0000x reference)
import jax
import jax.numpy as jnp
from jax.experimental import pallas as pl


def kernel(h_node, pos_node, h_edge, edge_index, node_time, edge_time, params):
    raise NotImplementedError("write your pallas kernel here")


# speedup vs baseline: 2.4196x; 2.4196x over previous
import functools
import numpy as np
import jax
import jax.numpy as jnp
from jax import lax
from jax.experimental import pallas as pl
from jax.experimental.pallas import tpu as pltpu, tpu_sc as plsc

N = 10000
E = 160000
ND = 128
ED = 16
CUTOFF = 10.0
NG = 16

NC = 2
NS = 16
CH = 128
NCHUNK = E // CH
ROWS_PER_TILE = N // NS

TE = 2000
TN = 2000

_MESH2 = dict(core_axis_name="c", subcore_axis_name="s", num_cores=2,
              num_subcores=NS)
_MESH1 = dict(core_axis_name="c", subcore_axis_name="s", num_cores=1,
              num_subcores=NS)


def _chunk_range(wid, nworkers):
    base = NCHUNK // nworkers
    rem = NCHUNK % nworkers
    start = wid * base + jnp.minimum(wid, rem)
    cnt = base + jnp.where(wid < rem, 1, 0)
    return start, cnt


def _make_gather(widths, sel_table, sel_left):
    n_t = len(widths)
    n_o = len(sel_table)
    out_type = tuple(
        jax.ShapeDtypeStruct((E, widths[sel_table[k]]), jnp.float32)
        for k in range(n_o))
    scratch = [pltpu.VMEM((CH,), jnp.int32), pltpu.VMEM((CH,), jnp.int32)]
    scratch += [pltpu.VMEM((CH, widths[sel_table[k]]), jnp.float32)
                for k in range(n_o)]
    scratch += [pltpu.SemaphoreType.DMA]

    def body(*refs):
        idxL = refs[0]
        idxR = refs[1]
        tables = refs[2:2 + n_t]
        outs = refs[2 + n_t:2 + n_t + n_o]
        ivL = refs[2 + n_t + n_o]
        ivR = refs[3 + n_t + n_o]
        bufs = refs[4 + n_t + n_o:4 + n_t + n_o + n_o]
        sem = refs[-1]
        wid = lax.axis_index("s") * NC + lax.axis_index("c")
        start, cnt = _chunk_range(wid, NC * NS)

        def chunk(j, carry):
            off = (start + j) * CH
            pltpu.sync_copy(idxL.at[pl.ds(off, CH)], ivL)
            pltpu.sync_copy(idxR.at[pl.ds(off, CH)], ivR)
            for k in range(n_o):
                iv = ivL if sel_left[k] else ivR
                pltpu.async_copy(tables[sel_table[k]].at[iv], bufs[k], sem).wait()
                pltpu.sync_copy(bufs[k], outs[k].at[pl.ds(off, CH)])
            return carry

        lax.fori_loop(0, cnt, chunk, 0)

    return pl.kernel(body, out_type=out_type,
                     mesh=plsc.VectorSubcoreMesh(**_MESH2),
                     compiler_params=pltpu.CompilerParams(
                         use_tc_tiling_on_sc=False),
                     scratch_types=scratch)


@functools.cache
def _gather_g1_k():
    return _make_gather([ND, ED], [0, 0, 1, 1], [True, False, True, False])


@functools.cache
def _gather_ns_k():
    return _make_gather([ED], [0, 0], [True, False])


@functools.cache
def _gather_g2_k():
    return _make_gather([ND, ED, ED], [0, 0, 1, 2], [True, False, True, False])


def _gather_g1(*a):
    return _gather_g1_k()(*a)


def _gather_ns(*a):
    return _gather_ns_k()(*a)


def _gather_g2(*a):
    return _gather_g2_k()(*a)


def _s1_body(idxL, idxR, msg, mL, mR, z128, z16,
             aggrP, sLP, sRP,
             ivL, ivR, bmsg, bmL, bmR, acc128, acc16L, acc16R):
    cid = lax.axis_index("c")
    sid = lax.axis_index("s")
    wid = sid * NC + cid
    r0 = sid * ROWS_PER_TILE
    pltpu.sync_copy(z128.at[pl.ds(r0, ROWS_PER_TILE)],
                    acc128.at[pl.ds(r0, ROWS_PER_TILE)])
    pltpu.sync_copy(z16.at[pl.ds(r0, ROWS_PER_TILE)],
                    acc16L.at[pl.ds(r0, ROWS_PER_TILE)])
    pltpu.sync_copy(z16.at[pl.ds(r0, ROWS_PER_TILE)],
                    acc16R.at[pl.ds(r0, ROWS_PER_TILE)])
    plsc.subcore_barrier()
    start, cnt = _chunk_range(wid, NC * NS)

    def chunk(j, carry):
        off = (start + j) * CH
        pltpu.sync_copy(idxL.at[pl.ds(off, CH)], ivL)
        pltpu.sync_copy(idxR.at[pl.ds(off, CH)], ivR)
        pltpu.sync_copy(msg.at[pl.ds(off, CH)], bmsg)
        pltpu.sync_copy(mL.at[pl.ds(off, CH)], bmL)
        pltpu.sync_copy(mR.at[pl.ds(off, CH)], bmR)
        pltpu.sync_copy(bmsg, acc128.at[ivL], add=True)
        pltpu.sync_copy(bmL, acc16L.at[ivR], add=True)
        pltpu.sync_copy(bmR, acc16R.at[ivL], add=True)
        return carry

    lax.fori_loop(0, cnt, chunk, 0)
    plsc.subcore_barrier()
    pltpu.sync_copy(acc128.at[pl.ds(r0, ROWS_PER_TILE)],
                    aggrP.at[cid, pl.ds(r0, ROWS_PER_TILE)])
    pltpu.sync_copy(acc16L.at[pl.ds(r0, ROWS_PER_TILE)],
                    sLP.at[cid, pl.ds(r0, ROWS_PER_TILE)])
    pltpu.sync_copy(acc16R.at[pl.ds(r0, ROWS_PER_TILE)],
                    sRP.at[cid, pl.ds(r0, ROWS_PER_TILE)])


@functools.cache
def _scatter_s1_k():
    return pl.kernel(
        _s1_body,
        out_type=(jax.ShapeDtypeStruct((NC, N, ND), jnp.float32),
                  jax.ShapeDtypeStruct((NC, N, ED), jnp.float32),
                  jax.ShapeDtypeStruct((NC, N, ED), jnp.float32)),
        mesh=plsc.VectorSubcoreMesh(**_MESH2),
        compiler_params=pltpu.CompilerParams(use_tc_tiling_on_sc=False),
        scratch_types=[
            pltpu.VMEM((CH,), jnp.int32), pltpu.VMEM((CH,), jnp.int32),
            pltpu.VMEM((CH, ND), jnp.float32),
            pltpu.VMEM((CH, ED), jnp.float32),
            pltpu.VMEM((CH, ED), jnp.float32),
            pltpu.VMEM_SHARED((N, ND), jnp.float32),
            pltpu.VMEM_SHARED((N, ED), jnp.float32),
            pltpu.VMEM_SHARED((N, ED), jnp.float32),
        ])


def _scatter_s1(*a):
    return _scatter_s1_k()(*a)


def _s2_body(idxL, force, ns_in, ns_out, ivL, bf, acc):
    sid = lax.axis_index("s")
    r0 = sid * ROWS_PER_TILE
    pltpu.sync_copy(ns_in.at[pl.ds(r0, ROWS_PER_TILE)],
                    acc.at[pl.ds(r0, ROWS_PER_TILE)])
    plsc.subcore_barrier()
    start, cnt = _chunk_range(sid, NS)

    def chunk(j, carry):
        off = (start + j) * CH
        pltpu.sync_copy(idxL.at[pl.ds(off, CH)], ivL)
        pltpu.sync_copy(force.at[pl.ds(off, CH)], bf)
        pltpu.sync_copy(bf, acc.at[ivL], add=True)
        return carry

    lax.fori_loop(0, cnt, chunk, 0)
    plsc.subcore_barrier()
    pltpu.sync_copy(acc.at[pl.ds(r0, ROWS_PER_TILE)],
                    ns_out.at[pl.ds(r0, ROWS_PER_TILE)])


@functools.cache
def _scatter_s2_k():
    return pl.kernel(
        _s2_body,
        out_type=jax.ShapeDtypeStruct((N, ED), jnp.float32),
        mesh=plsc.VectorSubcoreMesh(**_MESH1),
        compiler_params=pltpu.CompilerParams(use_tc_tiling_on_sc=False),
        scratch_types=[
            pltpu.VMEM((CH,), jnp.int32),
            pltpu.VMEM((CH, ED), jnp.float32),
            pltpu.VMEM_SHARED((N, ED), jnp.float32),
        ])


def _scatter_s2(*a):
    return _scatter_s2_k()(*a)


_G_STEP = np.float32(CUTOFF / (NG - 1))
_G_COEFF = np.float32(-0.5 / _G_STEP ** 2)


def _edge_spec(w):
    return pl.BlockSpec((TE, w), lambda i: (i, 0))


def _full_spec(a):
    nd = a.ndim
    return pl.BlockSpec(a.shape, lambda i, _n=nd: (0,) * _n)


def _ln(x, g, b):
    mu = jnp.mean(x, axis=-1, keepdims=True)
    var = jnp.mean((x - mu) * (x - mu), axis=-1, keepdims=True)
    return (x - mu) * lax.rsqrt(var + 1e-5) * g + b


def _rel_dist(nsl, nsr):
    rel = nsl[:, 0:3] - nsr[:, 0:3]
    d2 = jnp.sum(rel * rel, axis=-1, keepdims=True)
    dist = jnp.sqrt(d2)
    return rel, dist


def _tc_call(body, n_data, data, weights, out_widths):
    wlist = list(weights)
    in_specs = [_edge_spec(a.shape[1]) for a in data] + \
               [_full_spec(a) for a in wlist]
    out_shape = tuple(jax.ShapeDtypeStruct((E, w), jnp.float32)
                      for w in out_widths)
    out_specs = tuple(_edge_spec(w) for w in out_widths)
    return pl.pallas_call(
        body, grid=(E // TE,), in_specs=in_specs,
        out_specs=out_specs, out_shape=out_shape,
    )(*data, *wlist)


def _bond_ffn_tc(he, g, t, W):
    bf = he @ W['Wb']
    nf = g @ W['Wn']
    inter = jnp.maximum((bf * nf) @ W['Wi1'] + W['bi1'], 0.0) @ W['Wi2'] + W['bi2']
    gt = jnp.maximum(he @ W['G1a'] + g @ W['G1b'] + t * W['g1c'] + W['bg1'],
                     0.0) @ W['G2'] + W['bg2']
    return inter * jax.nn.sigmoid(gt)


def _kernel_a(names):
    def body(*refs):
        he_r, nsl_r, nsr_r, gl_r, gr_r, et_r = refs[:6]
        wr = dict(zip(names, refs[6:6 + len(names)]))
        msg_o, he_o, ml_o, mr_o, base_o = refs[6 + len(names):]
        w = lambda n: wr[n][...]

        he0 = he_r[...]
        nsl = nsl_r[...]
        nsr = nsr_r[...]
        gl = gl_r[...]
        gr = gr_r[...]
        et = et_r[...]
        _, dist = _rel_dist(nsl, nsr)
        offs = lax.broadcasted_iota(jnp.int32, (1, NG), 1).astype(
            jnp.float32) * _G_STEP
        gauss = jnp.exp(_G_COEFF * (dist - offs) ** 2)
        h_e = he0 @ w('emb0') + gauss @ w('emb1') + w('emb_b')
        he_o[...] = h_e

        hen = jnp.maximum(h_e @ w('en1') + w('en1b'), 0.0) @ w('en2') + w('en2b')
        hnn = jnp.maximum(gr @ w('nn1') + w('nn1b'), 0.0) @ w('nn2') + w('nn2b')
        m = (hen * hnn) @ w('mW') + w('mb')
        nt = nsr[:, 3:4]
        gt = jnp.maximum(h_e @ w('gg1a') + gr @ w('gg1b') + nt * w('gg1c')
                         + w('gg1bias'), 0.0) @ w('gg2') + w('gg2b')
        msg_o[...] = m * jax.nn.sigmoid(gt)

        WL = {k: w('L' + k) for k in
              ('Wb', 'Wn', 'Wi1', 'bi1', 'Wi2', 'bi2', 'G1a', 'G1b', 'g1c',
               'bg1', 'G2', 'bg2')}
        WR = {k: w('R' + k) for k in
              ('Wb', 'Wn', 'Wi1', 'bi1', 'Wi2', 'bi2', 'G1a', 'G1b', 'g1c',
               'bg1', 'G2', 'bg2')}
        ml_o[...] = _bond_ffn_tc(h_e, gl, et, WL)
        mr_o[...] = _bond_ffn_tc(h_e, gr, et, WR)
        base_o[...] = (gl @ w('fl') + w('flb') + gr @ w('fr') + w('frb')
                       + h_e @ w('sf') + w('sfb'))
    return body


def _kernel_b(names):
    def body(*refs):
        x_r, agP_r, sLP_r, sRP_r = refs[:4]
        wr = dict(zip(names, refs[4:4 + len(names)]))
        hnew_o, sL_o, sR_o = refs[4 + len(names):]
        w = lambda n: wr[n][...]
        x = x_r[...]
        ag = agP_r[...]
        aggr = ag[0] + ag[1]
        out = x @ w('cW') + w('cb') + aggr
        out = _ln(out, w('lng'), w('lnb'))
        hnew_o[...] = x + jnp.maximum(out, 0.0) @ w('oW') + w('ob')
        sL_o[...] = sLP_r[0] + sLP_r[1]
        sR_o[...] = sRP_r[0] + sRP_r[1]
    return body


def _kernel_c(names):
    def body(*refs):
        he_r, msgl_r, msgr_r, base_r, gln_r, grn_r, nsl_r, nsr_r, et_r = refs[:9]
        wr = dict(zip(names, refs[9:9 + len(names)]))
        henew_o, force_o = refs[9 + len(names):]
        w = lambda n: wr[n][...]
        h = msgl_r[...] + msgr_r[...] + base_r[...]
        h = _ln(h, w('elng'), w('elnb'))
        he_new = he_r[...] + jnp.maximum(h, 0.0) @ w('eoW') + w('eob')
        henew_o[...] = he_new

        gln = gln_r[...]
        grn = grn_r[...]
        et = et_r[...]
        lf = jnp.maximum(gln @ w('pl1') + w('pl1b'), 0.0) @ w('pl2') + w('pl2b')
        rf = jnp.maximum(grn @ w('pr1') + w('pr1b'), 0.0) @ w('pr2') + w('pr2b')
        pp = lf * rf
        PW = {k: w('P' + k) for k in
              ('Wb', 'Wn', 'Wi1', 'bi1', 'Wi2', 'bi2', 'G1a', 'G1b', 'g1c',
               'bg1', 'G2', 'bg2')}
        wgt = _bond_ffn_tc(he_new, pp, et, PW)
        rel, dist = _rel_dist(nsl_r[...], nsr_r[...])
        f3 = wgt * rel / dist / (dist + 1.0)
        force_o[...] = jnp.concatenate(
            [f3, jnp.zeros((f3.shape[0], ED - 3), jnp.float32)], axis=-1)
    return body


def _row(v):
    return v.reshape(1, -1)


def _bond_weights(prefix, p, gdin_main):
    g1 = p['gate']['l1']['W']
    return {
        prefix + 'Wb': p['bond_linear']['W'],
        prefix + 'Wn': p['node_linear']['W'],
        prefix + 'Wi1': p['inter']['l1']['W'],
        prefix + 'bi1': _row(p['inter']['l1']['b']),
        prefix + 'Wi2': p['inter']['l2']['W'],
        prefix + 'bi2': _row(p['inter']['l2']['b']),
        prefix + 'G1a': g1[:ED],
        prefix + 'G1b': g1[ED:ED + gdin_main],
        prefix + 'g1c': g1[ED + gdin_main:],
        prefix + 'bg1': _row(p['gate']['l1']['b']),
        prefix + 'G2': p['gate']['l2']['W'],
        prefix + 'bg2': _row(p['gate']['l2']['b']),
    }


def _weights_a(emb, nb, eb):
    gg1 = nb['gate']['l1']['W']
    d = {
        'emb0': emb['W'][:ED], 'emb1': emb['W'][ED:], 'emb_b': _row(emb['b']),
        'en1': nb['edge_net']['l1']['W'], 'en1b': _row(nb['edge_net']['l1']['b']),
        'en2': nb['edge_net']['l2']['W'], 'en2b': _row(nb['edge_net']['l2']['b']),
        'nn1': nb['node_net']['l1']['W'], 'nn1b': _row(nb['node_net']['l1']['b']),
        'nn2': nb['node_net']['l2']['W'], 'nn2b': _row(nb['node_net']['l2']['b']),
        'mW': nb['msg_net']['W'], 'mb': _row(nb['msg_net']['b']),
        'gg1a': gg1[:ED], 'gg1b': gg1[ED:ED + ND], 'gg1c': gg1[ED + ND:],
        'gg1bias': _row(nb['gate']['l1']['b']),
        'gg2': nb['gate']['l2']['W'], 'gg2b': _row(nb['gate']['l2']['b']),
        'fl': eb['node_ffn_left']['W'], 'flb': _row(eb['node_ffn_left']['b']),
        'fr': eb['node_ffn_right']['W'], 'frb': _row(eb['node_ffn_right']['b']),
        'sf': eb['self_ffn']['W'], 'sfb': _row(eb['self_ffn']['b']),
    }
    d.update(_bond_weights('L', eb['ffn_left'], ND))
    d.update(_bond_weights('R', eb['ffn_right'], ND))
    return d


def _weights_b(nb):
    return {
        'cW': nb['centroid']['W'], 'cb': _row(nb['centroid']['b']),
        'lng': _row(nb['ln']['g']), 'lnb': _row(nb['ln']['b']),
        'oW': nb['out']['W'], 'ob': _row(nb['out']['b']),
    }


def _weights_c(eb, pb):
    d = {
        'elng': _row(eb['ln']['g']), 'elnb': _row(eb['ln']['b']),
        'eoW': eb['out']['W'], 'eob': _row(eb['out']['b']),
        'pl1': pb['left']['l1']['W'], 'pl1b': _row(pb['left']['l1']['b']),
        'pl2': pb['left']['l2']['W'], 'pl2b': _row(pb['left']['l2']['b']),
        'pr1': pb['right']['l1']['W'], 'pr1b': _row(pb['right']['l1']['b']),
        'pr2': pb['right']['l2']['W'], 'pr2b': _row(pb['right']['l2']['b']),
    }
    d.update(_bond_weights('P', pb['edge_lin'], ED))
    return d


def kernel(h_node, pos_node, h_edge, edge_index, node_time, edge_time, params):
    idxL = edge_index[0].astype(jnp.int32)
    idxR = edge_index[1].astype(jnp.int32)
    ns = jnp.concatenate(
        [pos_node, node_time, jnp.zeros((N, ED - 4), jnp.float32)], axis=-1)
    z128 = jnp.zeros((N, ND), jnp.float32)
    z16 = jnp.zeros((N, ED), jnp.float32)

    h_cur = h_node
    he_cur = h_edge
    gl = gr = None

    for i in range(2):
        wa = _weights_a(params['edge_embs'][i], params['node_blocks'][i],
                        params['edge_blocks'][i])
        wb = _weights_b(params['node_blocks'][i])
        wc = _weights_c(params['edge_blocks'][i], params['pos_blocks'][i])

        if i == 0:
            gl, gr, nsl, nsr = _gather_g1(idxL, idxR, h_cur, ns)
        else:
            nsl, nsr = _gather_ns(idxL, idxR, ns)

        na = sorted(wa)
        msg, h_e, mL, mR, base = _tc_call(
            _kernel_a(na), 6, [he_cur, nsl, nsr, gl, gr, edge_time],
            [wa[k] for k in na], [ND, ED, ED, ED, ED])

        aggrP, sLP, sRP = _scatter_s1(idxL, idxR, msg, mL, mR, z128, z16)

        nb_names = sorted(wb)
        wlist = [wb[k] for k in nb_names]
        in_specs = ([pl.BlockSpec((TN, ND), lambda i_: (i_, 0)),
                     pl.BlockSpec((NC, TN, ND), lambda i_: (0, i_, 0)),
                     pl.BlockSpec((NC, TN, ED), lambda i_: (0, i_, 0)),
                     pl.BlockSpec((NC, TN, ED), lambda i_: (0, i_, 0))]
                    + [_full_spec(a) for a in wlist])
        h_new, sL, sR = pl.pallas_call(
            _kernel_b(nb_names), grid=(N // TN,), in_specs=in_specs,
            out_specs=(pl.BlockSpec((TN, ND), lambda i_: (i_, 0)),
                       pl.BlockSpec((TN, ED), lambda i_: (i_, 0)),
                       pl.BlockSpec((TN, ED), lambda i_: (i_, 0))),
            out_shape=(jax.ShapeDtypeStruct((N, ND), jnp.float32),
                       jax.ShapeDtypeStruct((N, ED), jnp.float32),
                       jax.ShapeDtypeStruct((N, ED), jnp.float32)),
        )(h_cur, aggrP, sLP, sRP, *wlist)

        gln, grn, msgL, msgR = _gather_g2(idxL, idxR, h_new, sL, sR)

        nc = sorted(wc)
        he_new, force = _tc_call(
            _kernel_c(nc), 9,
            [h_e, msgL, msgR, base, gln, grn, nsl, nsr, edge_time],
            [wc[k] for k in nc], [ED, ED])

        ns = _scatter_s2(idxL, force, ns)
        h_cur = h_new
        he_cur = he_new
        gl, gr = gln, grn

    return (h_cur, ns[:, :3], he_cur)
